# baseline (speedup 1.0000x reference)
import jax
import jax.numpy as jnp
from jax import lax
from jax.experimental import pallas as pl
from jax.experimental.pallas import tpu as pltpu
from jax.experimental.pallas import tpu_sc as plsc

NC = 2
NS = 16
LANES = 16
NW = NC * NS
EMBED = 64
CHUNK = 128
NBUF = 4

NROWS = 1000000
DT = EMBED // 8
SUPER = 512
SUB = 128
NSB = NROWS // SUPER
TAIL = NROWS - NSB * SUPER
SB_PER_W = 61
TNBUF = 2
ONBUF = 4


def _tr_body(src_hbm, tail_hbm, dst_hbm, buf0, buf1,
             obuf0, obuf1, obuf2, obuf3,
             isem0, isem1, osem0, osem1, osem2, osem3):
    buf = (buf0, buf1)
    obuf = (obuf0, obuf1, obuf2, obuf3)
    isem = (isem0, isem1)
    osem = (osem0, osem1, osem2, osem3)
    wid = lax.axis_index("s") * NC + lax.axis_index("c")

    def sb_col(i):
        return pl.multiple_of((wid + NW * i) * SUPER, SUPER)

    def fire_in(i, b):
        c0 = sb_col(i)
        for dt in range(DT):
            pltpu.async_copy(src_hbm.at[pl.ds(dt * 8, 8), pl.ds(c0, SUPER)],
                             buf[b].at[pl.ds(dt * 8, 8)], isem[b])

    def wait_in(b):
        for dt in range(DT):
            pltpu.make_async_copy(src_hbm.at[pl.ds(0, 8), pl.ds(0, SUPER)],
                                  buf[b].at[pl.ds(dt * 8, 8)], isem[b]).wait()

    def fire_out(i, sub, ob):
        c0 = sb_col(i) + sub * SUB
        pltpu.async_copy(obuf[ob],
                         dst_hbm.at[pl.ds(c0 * EMBED, SUB * EMBED)], osem[ob])

    def wait_out(ob):
        pltpu.make_async_copy(obuf[ob],
                              dst_hbm.at[pl.ds(0, SUB * EMBED)], osem[ob]).wait()

    iota = lax.iota(jnp.int32, LANES)
    base_vecs = [(jnp.full((LANES,), c * LANES, jnp.int32) + iota) * EMBED
                 for c in range(SUB // LANES)]

    def transpose_sub(b, sub, ob):
        def d_body(d, carry):
            for c in range(SUB // LANES):
                v = buf[b][d, pl.ds(sub * SUB + c * LANES, LANES)]
                plsc.store_scatter(obuf[ob], [base_vecs[c] + d], v)
            return carry

        lax.fori_loop(0, EMBED, d_body, 0, unroll=4)

    def process(i, b, first):
        wait_in(b)
        for sub in range(SUPER // SUB):
            if not first:
                wait_out(sub)
            transpose_sub(b, sub, sub)
            fire_out(i, sub, sub)

    fire_in(0, 0)
    fire_in(1, 1)
    process(0, 0, first=True)

    def step_body(i2, carry):
        i = 2 * i2 + 1

        @pl.when(i + 1 < SB_PER_W)
        def _():
            fire_in(i + 1, 0)

        process(i, 1, first=False)

        @pl.when(i + 2 < SB_PER_W)
        def _():
            fire_in(i + 2, 1)

        @pl.when(i + 1 < SB_PER_W)
        def _():
            process(i + 1, 0, first=False)

        return carry

    lax.fori_loop(0, SB_PER_W // 2, step_body, 0)
    for ob in range(ONBUF):
        wait_out(ob)

    @pl.when(wid == 0)
    def _():
        c0 = (NSB - 1) * SUPER
        for dt in range(DT):
            pltpu.sync_copy(src_hbm.at[pl.ds(dt * 8, 8), pl.ds(c0, SUPER)],
                            buf[0].at[pl.ds(dt * 8, 8)])
        for sub in range(SUPER // SUB):
            transpose_sub(0, sub, sub)
            pltpu.sync_copy(obuf[sub],
                            dst_hbm.at[pl.ds((c0 + sub * SUB) * EMBED,
                                             SUB * EMBED)])

    @pl.when(wid == 1)
    def _():
        c0 = NSB * SUPER
        pltpu.sync_copy(tail_hbm, obuf[0].at[pl.ds(0, TAIL * EMBED)])
        pltpu.sync_copy(obuf[0].at[pl.ds(0, TAIL * EMBED)],
                        dst_hbm.at[pl.ds(c0 * EMBED, TAIL * EMBED)])


def _mf_body(user_hbm, itemp_hbm, itemn_hbm, users_hbm, items_hbm,
             pscore_hbm, nscore_hbm,
             idxu, idxp, idxn, urows, prows, nrows, psc, nsc, *sems):
    gsem = sems[0:NBUF]
    isem = sems[NBUF:2 * NBUF]
    osem = sems[2 * NBUF:3 * NBUF]
    wid = lax.axis_index("s") * NC + lax.axis_index("c")
    n_per_w = user_hbm.shape[0] // NW
    nchunk = n_per_w // CHUNK
    base_w = wid * n_per_w

    def chunk_base(g):
        return pl.multiple_of(base_w + g * CHUNK, CHUNK)

    def fire_idx(g, b, sync):
        base = chunk_base(g)
        if sync:
            pltpu.sync_copy(user_hbm.at[pl.ds(base, CHUNK)], idxu.at[b])
            pltpu.sync_copy(itemp_hbm.at[pl.ds(base, CHUNK)], idxp.at[b])
            pltpu.sync_copy(itemn_hbm.at[pl.ds(base, CHUNK)], idxn.at[b])
        else:
            pltpu.async_copy(user_hbm.at[pl.ds(base, CHUNK)], idxu.at[b], isem[b])
            pltpu.async_copy(itemp_hbm.at[pl.ds(base, CHUNK)], idxp.at[b], isem[b])
            pltpu.async_copy(itemn_hbm.at[pl.ds(base, CHUNK)], idxn.at[b], isem[b])

    def wait_idx(b):
        pltpu.make_async_copy(user_hbm.at[pl.ds(0, CHUNK)], idxu.at[b], isem[b]).wait()
        pltpu.make_async_copy(itemp_hbm.at[pl.ds(0, CHUNK)], idxp.at[b], isem[b]).wait()
        pltpu.make_async_copy(itemn_hbm.at[pl.ds(0, CHUNK)], idxn.at[b], isem[b]).wait()

    def fire_gather(b):
        pltpu.async_copy(users_hbm.at[idxu.at[b]], urows.at[b], gsem[b])
        pltpu.async_copy(items_hbm.at[idxp.at[b]], prows.at[b], gsem[b])
        pltpu.async_copy(items_hbm.at[idxn.at[b]], nrows.at[b], gsem[b])

    def wait_gather(b):
        pltpu.make_async_copy(users_hbm.at[idxu.at[b]], urows.at[b], gsem[b]).wait()
        pltpu.make_async_copy(items_hbm.at[idxp.at[b]], prows.at[b], gsem[b]).wait()
        pltpu.make_async_copy(items_hbm.at[idxn.at[b]], nrows.at[b], gsem[b]).wait()

    def fire_out(g, b):
        base = chunk_base(g)
        pltpu.async_copy(psc.at[b], pscore_hbm.at[pl.ds(base, CHUNK)], osem[b])
        pltpu.async_copy(nsc.at[b], nscore_hbm.at[pl.ds(base, CHUNK)], osem[b])

    def wait_out(b):
        pltpu.make_async_copy(psc.at[b], pscore_hbm.at[pl.ds(0, CHUNK)], osem[b]).wait()
        pltpu.make_async_copy(nsc.at[b], nscore_hbm.at[pl.ds(0, CHUNK)], osem[b]).wait()

    def compute(b):
        last_lane = lax.iota(jnp.int32, LANES) == (LANES - 1)

        def row_body(r, carry):
            accp = None
            accn = None
            for k in range(EMBED // LANES):
                uu = urows[b, r, pl.ds(k * LANES, LANES)]
                pp = prows[b, r, pl.ds(k * LANES, LANES)]
                nn = nrows[b, r, pl.ds(k * LANES, LANES)]
                accp = uu * pp if accp is None else accp + uu * pp
                accn = uu * nn if accn is None else accn + uu * nn
            ridx = jnp.full((LANES,), 0, jnp.int32) + r
            plsc.store_scatter(psc.at[b], [ridx], plsc.cumsum(accp), mask=last_lane)
            plsc.store_scatter(nsc.at[b], [ridx], plsc.cumsum(accn), mask=last_lane)
            return carry

        lax.fori_loop(0, CHUNK, row_body, 0, unroll=4)

    for b in range(NBUF):
        fire_idx(b, b, sync=True)
        fire_gather(b)

    def outer(i, carry):
        for b in range(NBUF):
            g = i * NBUF + b
            wait_gather(b)

            @pl.when(g < nchunk - NBUF)
            def _():
                fire_idx(g + NBUF, b, sync=False)

            @pl.when(g >= NBUF)
            def _():
                wait_out(b)

            compute(b)
            fire_out(g, b)

            @pl.when(g < nchunk - NBUF)
            def _():
                wait_idx(b)
                fire_gather(b)

        return carry

    lax.fori_loop(0, nchunk // NBUF, outer, 0)
    for b in range(NBUF):
        wait_out(b)


def _transpose_table(table_t, tail_lin):
    mesh = plsc.VectorSubcoreMesh(core_axis_name="c", subcore_axis_name="s")
    f = pl.kernel(
        _tr_body,
        out_type=jax.ShapeDtypeStruct((NROWS * EMBED,), jnp.float32),
        mesh=mesh,
        compiler_params=pltpu.CompilerParams(needs_layout_passes=False,
                                             use_tc_tiling_on_sc=True),
        scratch_types=[
            pltpu.VMEM((EMBED, SUPER), jnp.float32),
            pltpu.VMEM((EMBED, SUPER), jnp.float32),
            pltpu.VMEM((SUB * EMBED,), jnp.float32),
            pltpu.VMEM((SUB * EMBED,), jnp.float32),
            pltpu.VMEM((SUB * EMBED,), jnp.float32),
            pltpu.VMEM((SUB * EMBED,), jnp.float32),
        ] + [pltpu.SemaphoreType.DMA] * 6,
    )
    return f(table_t, tail_lin)


def kernel(user, item_p, item_n, users, items):
    B, L = user.shape
    N = B * L
    uf = user.reshape(N)
    pf = item_p.reshape(N)
    nf = item_n.reshape(N)
    users_tail = users[NSB * SUPER:, :].reshape(TAIL * EMBED)
    items_tail = items[NSB * SUPER:, :].reshape(TAIL * EMBED)
    users_lin = _transpose_table(users.T, users_tail).reshape(NROWS, EMBED)
    items_lin = _transpose_table(items.T, items_tail).reshape(NROWS, EMBED)
    mesh = plsc.VectorSubcoreMesh(core_axis_name="c", subcore_axis_name="s")
    f = pl.kernel(
        _mf_body,
        out_type=(jax.ShapeDtypeStruct((N,), jnp.float32),
                  jax.ShapeDtypeStruct((N,), jnp.float32)),
        mesh=mesh,
        compiler_params=pltpu.CompilerParams(needs_layout_passes=False,
                                             use_tc_tiling_on_sc=False),
        scratch_types=[
            pltpu.VMEM((NBUF, CHUNK), jnp.int32),
            pltpu.VMEM((NBUF, CHUNK), jnp.int32),
            pltpu.VMEM((NBUF, CHUNK), jnp.int32),
            pltpu.VMEM((NBUF, CHUNK, EMBED), jnp.float32),
            pltpu.VMEM((NBUF, CHUNK, EMBED), jnp.float32),
            pltpu.VMEM((NBUF, CHUNK, EMBED), jnp.float32),
            pltpu.VMEM((NBUF, CHUNK), jnp.float32),
            pltpu.VMEM((NBUF, CHUNK), jnp.float32),
        ] + [pltpu.SemaphoreType.DMA] * (3 * NBUF),
    )
    p_score, n_score = f(uf, pf, nf, users_lin, items_lin)
    return p_score.reshape(B, L), n_score.reshape(B, L)


# speedup vs baseline: 1.5523x; 1.5523x over previous
import jax
import jax.numpy as jnp
from jax import lax
from jax.experimental import pallas as pl
from jax.experimental.pallas import tpu as pltpu
from jax.experimental.pallas import tpu_sc as plsc

NC = 2
NS = 16
LANES = 16
NW = NC * NS
EMBED = 64
CHUNK = 128
NBUF = 4


def _mf_body(user_hbm, itemp_hbm, itemn_hbm, users_hbm, items_hbm,
             pscore_hbm, nscore_hbm,
             idxu, idxp, idxn, urows, prows, nrows, psc, nsc, *sems):
    gsem = sems[0:NBUF]
    isem = sems[NBUF:2 * NBUF]
    osem = sems[2 * NBUF:3 * NBUF]
    wid = lax.axis_index("s") * NC + lax.axis_index("c")
    n_per_w = user_hbm.shape[0] // NW
    nchunk = n_per_w // CHUNK
    base_w = wid * n_per_w

    def chunk_base(g):
        return pl.multiple_of(base_w + g * CHUNK, CHUNK)

    def fire_idx(g, b, sync):
        base = chunk_base(g)
        if sync:
            pltpu.sync_copy(user_hbm.at[pl.ds(base, CHUNK)], idxu.at[b])
            pltpu.sync_copy(itemp_hbm.at[pl.ds(base, CHUNK)], idxp.at[b])
            pltpu.sync_copy(itemn_hbm.at[pl.ds(base, CHUNK)], idxn.at[b])
        else:
            pltpu.async_copy(user_hbm.at[pl.ds(base, CHUNK)], idxu.at[b], isem[b])
            pltpu.async_copy(itemp_hbm.at[pl.ds(base, CHUNK)], idxp.at[b], isem[b])
            pltpu.async_copy(itemn_hbm.at[pl.ds(base, CHUNK)], idxn.at[b], isem[b])

    def wait_idx(b):
        pltpu.make_async_copy(user_hbm.at[pl.ds(0, CHUNK)], idxu.at[b], isem[b]).wait()
        pltpu.make_async_copy(itemp_hbm.at[pl.ds(0, CHUNK)], idxp.at[b], isem[b]).wait()
        pltpu.make_async_copy(itemn_hbm.at[pl.ds(0, CHUNK)], idxn.at[b], isem[b]).wait()

    def fire_gather(b):
        pltpu.async_copy(users_hbm.at[idxu.at[b]], urows.at[b], gsem[b])
        pltpu.async_copy(items_hbm.at[idxp.at[b]], prows.at[b], gsem[b])
        pltpu.async_copy(items_hbm.at[idxn.at[b]], nrows.at[b], gsem[b])

    def wait_gather(b):
        pltpu.make_async_copy(users_hbm.at[idxu.at[b]], urows.at[b], gsem[b]).wait()
        pltpu.make_async_copy(items_hbm.at[idxp.at[b]], prows.at[b], gsem[b]).wait()
        pltpu.make_async_copy(items_hbm.at[idxn.at[b]], nrows.at[b], gsem[b]).wait()

    def fire_out(g, b):
        base = chunk_base(g)
        pltpu.async_copy(psc.at[b], pscore_hbm.at[pl.ds(base, CHUNK)], osem[b])
        pltpu.async_copy(nsc.at[b], nscore_hbm.at[pl.ds(base, CHUNK)], osem[b])

    def wait_out(b):
        pltpu.make_async_copy(psc.at[b], pscore_hbm.at[pl.ds(0, CHUNK)], osem[b]).wait()
        pltpu.make_async_copy(nsc.at[b], nscore_hbm.at[pl.ds(0, CHUNK)], osem[b]).wait()

    def compute(b):
        last_lane = lax.iota(jnp.int32, LANES) == (LANES - 1)

        def row_body(r, carry):
            accp = None
            accn = None
            for k in range(EMBED // (2 * LANES)):
                uu = plsc.unpack(urows[b, r, pl.ds(k * 2 * LANES, 2 * LANES)],
                                 format=plsc.PackFormat.INTERLEAVED)
                pp = plsc.unpack(prows[b, r, pl.ds(k * 2 * LANES, 2 * LANES)],
                                 format=plsc.PackFormat.INTERLEAVED)
                nn = plsc.unpack(nrows[b, r, pl.ds(k * 2 * LANES, 2 * LANES)],
                                 format=plsc.PackFormat.INTERLEAVED)
                for h in range(2):
                    accp = uu[h] * pp[h] if accp is None else accp + uu[h] * pp[h]
                    accn = uu[h] * nn[h] if accn is None else accn + uu[h] * nn[h]
            ridx = jnp.full((LANES,), 0, jnp.int32) + r
            plsc.store_scatter(psc.at[b], [ridx], plsc.cumsum(accp), mask=last_lane)
            plsc.store_scatter(nsc.at[b], [ridx], plsc.cumsum(accn), mask=last_lane)
            return carry

        lax.fori_loop(0, CHUNK, row_body, 0, unroll=4)

    for b in range(NBUF):
        fire_idx(b, b, sync=True)
        fire_gather(b)

    def outer(i, carry):
        for b in range(NBUF):
            g = i * NBUF + b
            wait_gather(b)

            @pl.when(g < nchunk - NBUF)
            def _():
                fire_idx(g + NBUF, b, sync=False)

            @pl.when(g >= NBUF)
            def _():
                wait_out(b)

            compute(b)
            fire_out(g, b)

            @pl.when(g < nchunk - NBUF)
            def _():
                wait_idx(b)
                fire_gather(b)

        return carry

    lax.fori_loop(0, nchunk // NBUF, outer, 0)

    for b in range(NBUF):
        wait_out(b)


def kernel(user, item_p, item_n, users, items):
    B, L = user.shape
    N = B * L
    uf = user.reshape(N)
    pf = item_p.reshape(N)
    nf = item_n.reshape(N)
    mesh = plsc.VectorSubcoreMesh(core_axis_name="c", subcore_axis_name="s")
    f = pl.kernel(
        _mf_body,
        out_type=(jax.ShapeDtypeStruct((N,), jnp.float32),
                  jax.ShapeDtypeStruct((N,), jnp.float32)),
        mesh=mesh,
        compiler_params=pltpu.CompilerParams(needs_layout_passes=False,
                                             use_tc_tiling_on_sc=False),
        scratch_types=[
            pltpu.VMEM((NBUF, CHUNK), jnp.int32),
            pltpu.VMEM((NBUF, CHUNK), jnp.int32),
            pltpu.VMEM((NBUF, CHUNK), jnp.int32),
            pltpu.VMEM((NBUF, CHUNK, EMBED), jnp.bfloat16),
            pltpu.VMEM((NBUF, CHUNK, EMBED), jnp.bfloat16),
            pltpu.VMEM((NBUF, CHUNK, EMBED), jnp.bfloat16),
            pltpu.VMEM((NBUF, CHUNK), jnp.float32),
            pltpu.VMEM((NBUF, CHUNK), jnp.float32),
        ] + [pltpu.SemaphoreType.DMA] * (3 * NBUF),
    )
    p_score, n_score = f(uf, pf, nf, users.astype(jnp.bfloat16),
                         items.astype(jnp.bfloat16))
    return p_score.reshape(B, L), n_score.reshape(B, L)
